# Initial kernel scaffold; baseline (speedup 1.0000x reference)
import jax
import jax.numpy as jnp
from jax import lax
from jax.experimental import pallas as pl
from jax.experimental.pallas import tpu as pltpu
from jax.experimental.pallas import tpu_sc as plsc

N_FEAT = 26
BATCH = 4096
HIST = 20
EMBED = 32

NUM_CORES = 2
NUM_SUBCORES = 16
NW = NUM_CORES * NUM_SUBCORES
B_PER_W = BATCH // NW
N_HALF = 2
B_HALF = B_PER_W // N_HALF
IDX_PER_HALF = B_HALF * HIST
GATHER_CHUNK = 128
N_CHUNKS = IDX_PER_HALF // GATHER_CHUNK


def _sc_body(ids_ref, *rest):
    tables = rest[:N_FEAT]
    out_ref = rest[N_FEAT]
    idx_v, rows_v, out_v, sem = rest[N_FEAT + 1:]

    w = lax.axis_index("s") * NUM_CORES + lax.axis_index("c")

    for f in range(N_FEAT):
        table = tables[f]

        def half_body(h, _, table=table, f=f):
            b0 = w * B_PER_W + h * B_HALF
            pltpu.sync_copy(ids_ref.at[f, w, h], idx_v)
            copies = []
            for c in range(N_CHUNKS):
                sl = pl.ds(c * GATHER_CHUNK, GATHER_CHUNK)
                copies.append(
                    pltpu.async_copy(table.at[idx_v.at[sl]], rows_v.at[sl], sem)
                )
            for cp in copies:
                cp.wait()

            def sample_body(b, _):
                base = b * HIST
                acc0 = rows_v[base, 0:16]
                acc1 = rows_v[base, 16:32]
                cnt = (idx_v[base] != 0).astype(jnp.int32)
                for l in range(1, HIST):
                    acc0 = acc0 + rows_v[base + l, 0:16]
                    acc1 = acc1 + rows_v[base + l, 16:32]
                    cnt = cnt + (idx_v[base + l] != 0).astype(jnp.int32)
                denom = jnp.maximum(cnt.astype(jnp.float32), 1.0)
                rcp = 1.0 / jnp.full((16,), denom, jnp.float32)
                out_v[b, 0:16] = acc0 * rcp
                out_v[b, 16:32] = acc1 * rcp
                return ()

            lax.fori_loop(0, B_HALF, sample_body, ())
            pltpu.sync_copy(out_v, out_ref.at[f, pl.ds(b0, B_HALF)])
            return ()

        lax.fori_loop(0, N_HALF, half_body, ())


def _pooled_all(ids_stacked, tables):
    kfn = pl.kernel(
        _sc_body,
        out_type=jax.ShapeDtypeStruct((N_FEAT, BATCH, EMBED), jnp.float32),
        mesh=plsc.VectorSubcoreMesh(core_axis_name="c", subcore_axis_name="s"),
        scratch_types=[
            pltpu.VMEM((IDX_PER_HALF,), jnp.int32),
            pltpu.VMEM((IDX_PER_HALF, EMBED), jnp.float32),
            pltpu.VMEM((B_HALF, EMBED), jnp.float32),
            pltpu.SemaphoreType.DMA,
        ],
    )
    return kfn(ids_stacked, *tables)


def kernel(*args):
    ids = args[:N_FEAT]
    tables = args[N_FEAT:]
    ids_stacked = jnp.stack(ids).reshape(N_FEAT, NW, N_HALF, IDX_PER_HALF)
    pooled = _pooled_all(ids_stacked, tables)
    return tuple(pooled[i] for i in range(N_FEAT))


# speedup vs baseline: 3.3140x; 3.3140x over previous
import jax
import jax.numpy as jnp
from jax import lax
from jax.experimental import pallas as pl
from jax.experimental.pallas import tpu as pltpu
from jax.experimental.pallas import tpu_sc as plsc

N_FEAT = 26
BATCH = 4096
HIST = 20
EMBED = 32

NUM_CORES = 2
NUM_SUBCORES = 16
NW = NUM_CORES * NUM_SUBCORES
B_PER_W = BATCH // NW
IDX_PER_W = B_PER_W * HIST
GATHER_CHUNK = 128
N_CHUNKS = IDX_PER_W // GATHER_CHUNK


def _sc_body(ids_ref, *rest):
    tables = rest[:N_FEAT]
    out_ref = rest[N_FEAT]
    idx_v, rows_v, out_v, cnt_v, sem = rest[N_FEAT + 1:]

    w = lax.axis_index("s") * NUM_CORES + lax.axis_index("c")
    lanes = lax.iota(jnp.int32, 16)

    def feature_body(f, _):
        pltpu.sync_copy(ids_ref.at[f, w], idx_v)

        for fs in range(N_FEAT):
            @pl.when(f == fs)
            def _(fs=fs):
                def fire(c, _):
                    sl = pl.ds(c * GATHER_CHUNK, GATHER_CHUNK)
                    pltpu.async_copy(
                        tables[fs].at[idx_v.at[sl]], rows_v.at[sl], sem
                    )
                    return ()
                lax.fori_loop(0, N_CHUNKS, fire, ())

        def count(g, _):
            sample_base = (g * 16 + lanes) * HIST
            acc = jnp.zeros((16,), jnp.float32)
            for l in range(HIST):
                vals = plsc.load_gather(idx_v, [sample_base + l])
                acc = acc + jnp.where(vals != 0, 1.0, 0.0).astype(jnp.float32)
            cnt_v[pl.ds(g * 16, 16)] = 1.0 / jnp.maximum(acc, 1.0)
            return ()
        lax.fori_loop(0, B_PER_W // 16, count, ())

        def drain(c, _):
            sl = pl.ds(c * GATHER_CHUNK, GATHER_CHUNK)
            pltpu.make_async_copy(
                tables[0].at[idx_v.at[sl]], rows_v.at[sl], sem
            ).wait()
            return ()
        lax.fori_loop(0, N_CHUNKS, drain, ())

        def sample_body(b, _):
            base = b * HIST
            acc0 = rows_v[base, 0:16]
            acc1 = rows_v[base, 16:32]
            for l in range(1, HIST):
                acc0 = acc0 + rows_v[base + l, 0:16]
                acc1 = acc1 + rows_v[base + l, 16:32]
            rcp = plsc.load_gather(cnt_v, [jnp.full((16,), b, jnp.int32)])
            out_v[b, 0:16] = acc0 * rcp
            out_v[b, 16:32] = acc1 * rcp
            return ()
        lax.fori_loop(0, B_PER_W, sample_body, ())

        pltpu.sync_copy(out_v, out_ref.at[f, pl.ds(w * B_PER_W, B_PER_W)])
        return ()

    lax.fori_loop(0, N_FEAT, feature_body, ())


def _pooled_all(ids_stacked, tables):
    kfn = pl.kernel(
        _sc_body,
        out_type=jax.ShapeDtypeStruct((N_FEAT, BATCH, EMBED), jnp.float32),
        mesh=plsc.VectorSubcoreMesh(core_axis_name="c", subcore_axis_name="s"),
        compiler_params=pltpu.CompilerParams(
            needs_layout_passes=False, use_tc_tiling_on_sc=False
        ),
        scratch_types=[
            pltpu.VMEM((IDX_PER_W,), jnp.int32),
            pltpu.VMEM((IDX_PER_W, EMBED), jnp.float32),
            pltpu.VMEM((B_PER_W, EMBED), jnp.float32),
            pltpu.VMEM((B_PER_W,), jnp.float32),
            pltpu.SemaphoreType.DMA,
        ],
    )
    return kfn(ids_stacked, *tables)


def kernel(*args):
    ids = args[:N_FEAT]
    tables = args[N_FEAT:]
    ids_stacked = jnp.stack(ids).reshape(N_FEAT, NW, IDX_PER_W)
    pooled = _pooled_all(ids_stacked, tables)
    return tuple(pooled[i] for i in range(N_FEAT))
